# baseline (speedup 1.0000x reference)
import jax
import jax.numpy as jnp
from jax import lax
from jax.experimental import pallas as pl
from jax.experimental.pallas import tpu as pltpu
from jax.experimental.pallas import tpu_sc as plsc

_NUM_BUCKETS = 1000
_D = 64
_NC, _NS, _L = 2, 16, 16
_NW = _NC * _NS
_B = 128
_W = _D // 2


def _qr_body(idx_hbm, t_hbm, out_hbm,
             idx_t, tbl, qt, rt, ova, ovb, ssa, ssb):
    wid = lax.axis_index("s") * _NC + lax.axis_index("c")
    f = idx_hbm.shape[0]
    nb_rows = idx_hbm.shape[1]
    rows_w = nb_rows // _NW
    blks = rows_w // _B
    bshift = blks.bit_length() - 1
    n_chunks = f * blks
    nb = jnp.full((_L,), _NUM_BUCKETS, jnp.int32)
    wsplat = jnp.full((_L,), _W, jnp.int32)

    ov = (ova, ovb)
    sem_s = (ssa, ssb)

    pltpu.sync_copy(t_hbm, tbl)
    pltpu.sync_copy(idx_hbm.at[:, pl.ds(wid * rows_w, rows_w)], idx_t)

    @pl.loop(0, n_chunks, step=2)
    def pipe(c0):
        for b in range(2):
            c = c0 + b
            fld = lax.shift_right_logical(c, bshift)
            blk = lax.bitwise_and(c, blks - 1)

            for m in range(_B // _L):
                v = idx_t[fld, pl.ds(blk * _B + m * _L, _L)]
                qt[pl.ds(m * _L, _L)] = lax.div(v, nb) * wsplat
                rt[pl.ds(m * _L, _L)] = (lax.rem(v, nb) + nb) * wsplat

            @pl.when(c >= 2)
            def _():
                pltpu.make_async_copy(
                    ov[b], out_hbm.at[0, :, pl.ds(0, _B)], sem_s[b]).wait()

            for m in range(_B // _L):
                qv = qt[pl.ds(m * _L, _L)]
                rv = rt[pl.ds(m * _L, _L)]

                @plsc.parallel_loop(0, _W, unroll=4)
                def col_body(k):
                    kk = jnp.zeros((_L,), jnp.int32) + k
                    qw = plsc.load_gather(tbl, [qv + kk])
                    rw = plsc.load_gather(tbl, [rv + kk])
                    qa, qb = plsc.unpack(
                        plsc.bitcast(qw, jnp.bfloat16),
                        format=plsc.PackFormat.INTERLEAVED)
                    ra, rb = plsc.unpack(
                        plsc.bitcast(rw, jnp.bfloat16),
                        format=plsc.PackFormat.INTERLEAVED)
                    s = pl.ds(m * _L, _L)
                    ov[b][2 * k, s] = qa * ra
                    ov[b][2 * k + 1, s] = qb * rb

            pltpu.async_copy(
                ov[b],
                out_hbm.at[fld, :, pl.ds(wid * rows_w + blk * _B, _B)],
                sem_s[b])

    pltpu.make_async_copy(ova, out_hbm.at[0, :, pl.ds(0, _B)], ssa).wait()
    pltpu.make_async_copy(ovb, out_hbm.at[0, :, pl.ds(0, _B)], ssb).wait()


def kernel(inputs, q_table, r_table):
    b, f = inputs.shape
    idx_t = inputs.T
    t_stack = jnp.concatenate([q_table, r_table], axis=0).astype(jnp.bfloat16)
    t_packed = lax.bitcast_convert_type(
        t_stack.reshape(2 * _NUM_BUCKETS, _W, 2), jnp.int32
    ).reshape(2 * _NUM_BUCKETS * _W)
    mesh = plsc.VectorSubcoreMesh(core_axis_name="c", subcore_axis_name="s")
    out_phys = pl.kernel(
        _qr_body,
        mesh=mesh,
        compiler_params=pltpu.CompilerParams(needs_layout_passes=False),
        out_type=jax.ShapeDtypeStruct((f, _D, b), jnp.float32),
        scratch_types=(
            [pltpu.VMEM((f, b // _NW), jnp.int32)]
            + [pltpu.VMEM((2 * _NUM_BUCKETS * _W,), jnp.int32)]
            + [pltpu.VMEM((_B,), jnp.int32)] * 2
            + [pltpu.VMEM((_D, _B), jnp.float32)] * 2
            + [pltpu.SemaphoreType.DMA] * 2
        ),
    )(idx_t, t_packed)
    return jnp.transpose(out_phys, (2, 0, 1))


# speedup vs baseline: 2.4495x; 2.4495x over previous
import jax
import jax.numpy as jnp
from jax import lax
from jax.experimental import pallas as pl
from jax.experimental.pallas import tpu as pltpu
from jax.experimental.pallas import tpu_sc as plsc

_NUM_BUCKETS = 1000
_D = 64
_NC, _NS, _L = 2, 16, 16
_NW = _NC * _NS
_B = 128
_W = _D // 2
_WS = _W + 1


def _qr_body(idx_hbm, t_hbm, out_hbm,
             idx_t, tbl, qt, rt, ova, ovb, ssa, ssb):
    wid = lax.axis_index("s") * _NC + lax.axis_index("c")
    f = idx_hbm.shape[0]
    nb_rows = idx_hbm.shape[1]
    rows_w = nb_rows // _NW
    blks = rows_w // _B
    bshift = blks.bit_length() - 1
    n_chunks = f * blks
    nb = jnp.full((_L,), _NUM_BUCKETS, jnp.int32)
    wsplat = jnp.full((_L,), _WS, jnp.int32)
    hmask = jnp.full((_L,), -65536, jnp.int32)
    sh16 = jnp.full((_L,), 16, jnp.int32)

    ov = (ova, ovb)
    sem_s = (ssa, ssb)

    pltpu.sync_copy(t_hbm, tbl)
    pltpu.sync_copy(idx_hbm.at[:, pl.ds(wid * rows_w, rows_w)], idx_t)

    @pl.loop(0, n_chunks, step=2)
    def pipe(c0):
        for b in range(2):
            c = c0 + b
            fld = lax.shift_right_logical(c, bshift)
            blk = lax.bitwise_and(c, blks - 1)

            for m in range(_B // _L):
                v = idx_t[fld, pl.ds(blk * _B + m * _L, _L)]
                qt[pl.ds(m * _L, _L)] = lax.div(v, nb) * wsplat
                rt[pl.ds(m * _L, _L)] = (lax.rem(v, nb) + nb) * wsplat

            @pl.when(c >= 2)
            def _():
                pltpu.make_async_copy(
                    ov[b], out_hbm.at[0, :, pl.ds(0, _B)], sem_s[b]).wait()

            for m in range(_B // _L):
                qv = qt[pl.ds(m * _L, _L)]
                rv = rt[pl.ds(m * _L, _L)]

                @plsc.parallel_loop(0, _W, unroll=4)
                def col_body(k):
                    kk = jnp.zeros((_L,), jnp.int32) + k
                    qw = plsc.load_gather(tbl, [qv + kk])
                    rw = plsc.load_gather(tbl, [rv + kk])
                    qa = plsc.bitcast(lax.shift_left(qw, sh16), jnp.float32)
                    qb = plsc.bitcast(lax.bitwise_and(qw, hmask), jnp.float32)
                    ra = plsc.bitcast(lax.shift_left(rw, sh16), jnp.float32)
                    rb = plsc.bitcast(lax.bitwise_and(rw, hmask), jnp.float32)
                    s = pl.ds(m * _L, _L)
                    ov[b][2 * k, s] = qa * ra
                    ov[b][2 * k + 1, s] = qb * rb

            pltpu.async_copy(
                ov[b],
                out_hbm.at[fld, :, pl.ds(wid * rows_w + blk * _B, _B)],
                sem_s[b])

    pltpu.make_async_copy(ova, out_hbm.at[0, :, pl.ds(0, _B)], ssa).wait()
    pltpu.make_async_copy(ovb, out_hbm.at[0, :, pl.ds(0, _B)], ssb).wait()


def kernel(inputs, q_table, r_table):
    b, f = inputs.shape
    idx_t = inputs.T
    t_stack = jnp.concatenate([q_table, r_table], axis=0).astype(jnp.bfloat16)
    t_packed = jnp.pad(
        lax.bitcast_convert_type(
            t_stack.reshape(2 * _NUM_BUCKETS, _W, 2), jnp.int32),
        ((0, 0), (0, _WS - _W))).reshape(2 * _NUM_BUCKETS * _WS)
    mesh = plsc.VectorSubcoreMesh(core_axis_name="c", subcore_axis_name="s")
    out_phys = pl.kernel(
        _qr_body,
        mesh=mesh,
        compiler_params=pltpu.CompilerParams(needs_layout_passes=False),
        out_type=jax.ShapeDtypeStruct((f, _D, b), jnp.float32),
        scratch_types=(
            [pltpu.VMEM((f, b // _NW), jnp.int32)]
            + [pltpu.VMEM((2 * _NUM_BUCKETS * _WS,), jnp.int32)]
            + [pltpu.VMEM((_B,), jnp.int32)] * 2
            + [pltpu.VMEM((_D, _B), jnp.float32)] * 2
            + [pltpu.SemaphoreType.DMA] * 2
        ),
    )(idx_t, t_packed)
    return jnp.transpose(out_phys, (2, 0, 1))
